# baseline (speedup 1.0000x reference)
import jax
import jax.numpy as jnp
import numpy as np
from jax.experimental import pallas as pl
from jax.experimental.pallas import tpu as pltpu

ROWS = 128
COLS = 100000
BR = 8
BC = 8192
CH = 512
NCH = BC // CH
NCB = (COLS + BC - 1) // BC

_U32 = jnp.uint32
_TINY = np.float32(np.finfo(np.float32).tiny)
_NEG_INF = np.float32(-np.inf)
_LN2 = np.float32(0.6931471805599453)


def _threefry_xor_fold(x1):
    k0 = np.uint32(0)
    k1 = np.uint32(42)
    ks = (k0, k1, np.uint32(k0 ^ k1 ^ np.uint32(0x1BD11BDA)))
    rot = ((13, 15, 26, 6), (17, 29, 16, 24))

    x0 = jnp.zeros_like(x1)
    for n in range(5):
        for r in rot[n % 2]:
            x0 = x0 + x1
            x1 = (x1 << _U32(r)) | (x1 >> _U32(32 - r))
            x1 = x1 ^ x0
        x0 = x0 + ks[(n + 1) % 3]
        x1 = x1 + ks[(n + 2) % 3] + _U32(n + 1)
    return x0 ^ x1


def _gumbel_from_bits(bits):
    fl = jax.lax.bitcast_convert_type(
        (bits >> _U32(9)) | _U32(0x3F800000), jnp.float32
    )
    u = fl - np.float32(1.0)
    u = jnp.maximum(_TINY, u * (np.float32(1.0) - _TINY) + _TINY)
    return -jnp.log(-jnp.log(u))


def _body(x_ref, out_ref, bestv, besti):
    r = pl.program_id(0)
    c = pl.program_id(1)

    lane = jax.lax.broadcasted_iota(jnp.int32, (BR, CH), 1)
    row = r * BR + jax.lax.broadcasted_iota(jnp.int32, (BR, CH), 0)
    base42 = (row * COLS + lane + 42).astype(_U32)

    init_v = jnp.full((BR, CH), _NEG_INF, jnp.float32)
    init_i = jnp.zeros((BR, CH), jnp.int32)

    def chunk(j, carry):
        bv, bi = carry
        off = c * BC + j * CH
        x1 = base42 + off.astype(_U32)
        g = _gumbel_from_bits(_threefry_xor_fold(x1))
        xv = x_ref[:, pl.ds(pl.multiple_of(j * CH, CH), CH)]
        val = xv + g
        col = lane + off
        val = jnp.where(col < COLS, val, _NEG_INF)
        upd = val > bv
        bv = jnp.where(upd, val, bv)
        bi = jnp.where(upd, col, bi)
        return bv, bi

    start = (
        jnp.where(c == 0, init_v, bestv[...]),
        jnp.where(c == 0, init_i, besti[...]),
    )
    bv, bi = jax.lax.fori_loop(0, NCH, chunk, start)
    bestv[...] = bv
    besti[...] = bi

    @pl.when(c == NCB - 1)
    def _emit():
        m = jnp.max(bv, axis=1, keepdims=True)
        cand = jnp.where(bv == m, bi, jnp.int32(COLS))
        out_ref[...] = jnp.min(cand, axis=1, keepdims=True)


@jax.jit
def _run(outputs):
    out = pl.pallas_call(
        _body,
        grid=(ROWS // BR, NCB),
        in_specs=[pl.BlockSpec((BR, BC), lambda r, c: (r, c))],
        out_specs=pl.BlockSpec((BR, 1), lambda r, c: (r, 0)),
        out_shape=jax.ShapeDtypeStruct((ROWS, 1), jnp.int32),
        scratch_shapes=[
            pltpu.VMEM((BR, CH), jnp.float32),
            pltpu.VMEM((BR, CH), jnp.int32),
        ],
        compiler_params=pltpu.CompilerParams(
            dimension_semantics=("parallel", "arbitrary"),
        ),
    )(outputs)
    return out[:, 0]


def kernel(outputs):
    return _run(outputs)
